# baseline (speedup 1.0000x reference)
import functools

import jax
import jax.numpy as jnp
import numpy as np
from jax import lax
from jax.experimental import pallas as pl
from jax.experimental.pallas import tpu as pltpu
from jax.experimental.pallas import tpu_sc as plsc

N_NODES = 50000
N_PAD = 50048
STRIPE = N_PAD // 16
HALF = 32
NC, NS = 2, 16

E_PAD = 802816
E_ROWS = E_PAD // 128
SP_TROWS = E_ROWS // 16
SP_CH = 2
SP_NBLK = SP_TROWS // SP_CH
DG_WROWS = E_ROWS // 32
DG_BLKS = DG_WROWS // 4

BATCH = 4096
R_BLK = 2048
N_BLK = 25

_MESH = plsc.VectorSubcoreMesh(
    core_axis_name="c", subcore_axis_name="s", num_cores=NC, num_subcores=NS)

def _shuffle_bf16(t):
    r = t.shape[0]
    return (t.reshape(r, 2, 16).transpose(0, 2, 1)
            .reshape(r, HALF).astype(jnp.bfloat16))


def _zero_rows(buf, n_rows, stripe_base, acc):
    def zb(e, _):
        buf[e, pl.ds(0, 16)] = jnp.zeros((16,), jnp.float32)
        buf[e, pl.ds(16, 16)] = jnp.zeros((16,), jnp.float32)
        return 0
    lax.fori_loop(0, n_rows, zb, 0)
    full, rem = STRIPE // n_rows, STRIPE % n_rows
    for k in range(full):
        pltpu.sync_copy(buf.at[pl.ds(0, n_rows)],
                        acc.at[pl.ds(stripe_base + k * n_rows, n_rows)])
    if rem:
        pltpu.sync_copy(buf.at[pl.ds(0, rem)],
                        acc.at[pl.ds(stripe_base + full * n_rows, rem)])


def _zero_1d(buf, n, stripe_base, acc):
    def zb(i, _):
        buf[pl.ds(i * 16, 16)] = jnp.zeros((16,), jnp.float32)
        return 0
    lax.fori_loop(0, n // 16, zb, 0)
    full, rem = STRIPE // n, STRIPE % n
    for k in range(full):
        pltpu.sync_copy(buf.at[pl.ds(0, n)],
                        acc.at[pl.ds(stripe_base + k * n, n)])
    if rem:
        pltpu.sync_copy(buf.at[pl.ds(0, rem)],
                        acc.at[pl.ds(stripe_base + full * n, rem)])


def _deg_body(dst2d, ew1d, out_p0, out_p1, acc, dbuf, ewbuf, zbuf):
    c = lax.axis_index("c")
    s = lax.axis_index("s")
    _zero_1d(zbuf, 512, s * STRIPE, acc)
    plsc.subcore_barrier()
    w = c * NS + s

    def blk(b, _):
        row0 = w * DG_WROWS + b * 4
        pltpu.sync_copy(dst2d.at[pl.ds(row0, 4)], dbuf)
        pltpu.sync_copy(ew1d.at[pl.ds(row0 * 128, 512)], ewbuf)
        for j in range(4):
            pltpu.sync_copy(ewbuf.at[pl.ds(j * 128, 128)],
                            acc.at[dbuf.at[j]], add=True)
        return 0

    lax.fori_loop(0, DG_BLKS, blk, 0)
    plsc.subcore_barrier()

    def drain(out):
        full, rem = STRIPE // 512, STRIPE % 512
        for k in range(full + (1 if rem else 0)):
            n = 512 if k < full else rem
            off = s * STRIPE + k * 512
            pltpu.sync_copy(acc.at[pl.ds(off, n)], zbuf.at[pl.ds(0, n)])
            pltpu.sync_copy(zbuf.at[pl.ds(0, n)], out.at[pl.ds(off, n)])

    @pl.when(c == 0)
    def _():
        drain(out_p0)

    @pl.when(c == 1)
    def _():
        drain(out_p1)


_deg_call = pl.kernel(
    _deg_body,
    out_type=[jax.ShapeDtypeStruct((N_PAD,), jnp.float32),
              jax.ShapeDtypeStruct((N_PAD,), jnp.float32)],
    mesh=_MESH,
    scratch_types=[
        pltpu.VMEM_SHARED((N_PAD,), jnp.float32),
        pltpu.VMEM((4, 128), jnp.int32),
        pltpu.VMEM((512,), jnp.float32),
        pltpu.VMEM((512,), jnp.float32),
    ],
)


def _spmm_body(tab_a, tab_b, src2d, dst2d, ew1d, out_a, out_b,
               acc, sbuf, dbuf, ewbuf, rbuf, rbf, isem, gsem, ssem):
    c = lax.axis_index("c")
    s = lax.axis_index("s")
    _zero_rows(rbuf.at[0], SP_CH * 128, s * STRIPE, acc)
    plsc.subcore_barrier()
    NB = SP_NBLK

    def stage(b, p, q):
        r0 = s * SP_TROWS + b * SP_CH
        pltpu.async_copy(src2d.at[pl.ds(r0, SP_CH)], sbuf.at[p], isem)
        pltpu.async_copy(dst2d.at[pl.ds(r0, SP_CH)], dbuf.at[q], isem)
        pltpu.async_copy(ew1d.at[pl.ds(r0 * 128, SP_CH * 128)],
                         ewbuf.at[p], isem)

    def wait_stage(p, q):
        pltpu.make_async_copy(src2d.at[pl.ds(0, SP_CH)], sbuf.at[p],
                              isem).wait()
        pltpu.make_async_copy(dst2d.at[pl.ds(0, SP_CH)], dbuf.at[q],
                              isem).wait()
        pltpu.make_async_copy(ew1d.at[pl.ds(0, SP_CH * 128)], ewbuf.at[p],
                              isem).wait()

    def run(tab, out):
        def fire(p):
            for j in range(SP_CH):
                pltpu.async_copy(tab.at[sbuf.at[p, j]],
                                 rbf.at[p, pl.ds(j * 128, 128)], gsem)

        def wait_fire(p):
            for j in range(SP_CH):
                pltpu.make_async_copy(tab.at[sbuf.at[p, j]],
                                      rbf.at[p, pl.ds(j * 128, 128)],
                                      gsem).wait()

        def scale(p):
            def body(g, _):
                wv = ewbuf[p, pl.ds(g * 16, 16)]
                for k in range(16):
                    w = wv[k]
                    e = g * 16 + k
                    lo, hi = plsc.unpack(
                        rbf[p, e, :], format=plsc.PackFormat.INTERLEAVED,
                        preferred_element_type=jnp.float32)
                    rbuf[p, e, pl.ds(0, 16)] = lo * w
                    rbuf[p, e, pl.ds(16, 16)] = hi * w
                return 0
            lax.fori_loop(0, SP_CH * 8, body, 0)

        def scat(p, q):
            for j in range(SP_CH):
                pltpu.async_copy(rbuf.at[p, pl.ds(j * 128, 128)],
                                 acc.at[dbuf.at[q, j]], ssem, add=True)

        def wait_scat(p, q):
            for j in range(SP_CH):
                pltpu.make_async_copy(rbuf.at[p, pl.ds(j * 128, 128)],
                                      acc.at[dbuf.at[q, j]], ssem).wait()

        stage(0, 0, 0)
        wait_stage(0, 0)
        fire(0)
        stage(1, 1, 1)

        def outer(bb, _):
            for q in range(4):
                b = bb * 4 + q
                p = q % 2

                @pl.when(b + 1 < NB)
                def _():
                    wait_stage(p ^ 1, (q + 1) % 4)

                @pl.when(b > 0)
                def _():
                    wait_scat(p ^ 1, (q - 1) % 4)

                @pl.when(b + 1 < NB)
                def _():
                    fire(p ^ 1)
                wait_fire(p)
                scale(p)
                scat(p, q)

                @pl.when(b + 2 < NB)
                def _():
                    stage(b + 2, p, (q + 2) % 4)
            return 0

        lax.fori_loop(0, NB // 4, outer, 0)
        wait_scat(1, 3)
        plsc.subcore_barrier()
        nbuf = SP_CH * 128
        full, rem = STRIPE // nbuf, STRIPE % nbuf
        for k in range(full + (1 if rem else 0)):
            n = nbuf if k < full else rem
            off = s * STRIPE + k * nbuf
            pltpu.sync_copy(acc.at[pl.ds(off, n)], rbuf.at[0, pl.ds(0, n)])
            pltpu.sync_copy(rbuf.at[0, pl.ds(0, n)], out.at[pl.ds(off, n)])

    @pl.when(c == 0)
    def _():
        run(tab_a, out_a)

    @pl.when(c == 1)
    def _():
        run(tab_b, out_b)


_spmm_call = pl.kernel(
    _spmm_body,
    out_type=[jax.ShapeDtypeStruct((N_PAD, HALF), jnp.float32),
              jax.ShapeDtypeStruct((N_PAD, HALF), jnp.float32)],
    mesh=_MESH,
    scratch_types=[
        pltpu.VMEM_SHARED((N_PAD, HALF), jnp.float32),
        pltpu.VMEM((2, SP_CH, 128), jnp.int32),
        pltpu.VMEM((4, SP_CH, 128), jnp.int32),
        pltpu.VMEM((2, SP_CH * 128), jnp.float32),
        pltpu.VMEM((2, SP_CH * 128, HALF), jnp.float32),
        pltpu.VMEM((2, SP_CH * 128, HALF), jnp.bfloat16),
        pltpu.SemaphoreType.DMA,
        pltpu.SemaphoreType.DMA,
        pltpu.SemaphoreType.DMA,
    ],
    compiler_params=pltpu.CompilerParams(use_tc_tiling_on_sc=False, needs_layout_passes=False),
)


def _gath_body(acc_a, acc_b, t2a, t2b, dis, bvec, drug2d, z_out,
               ibuf, ga, gb, ta, tb, db, bb, zbuf, sem):
    c = lax.axis_index("c")
    s = lax.axis_index("s")
    w = c * NS + s
    pltpu.sync_copy(drug2d.at[w], ibuf)
    pltpu.sync_copy(bvec, bb)
    descs = [
        pltpu.async_copy(acc_a.at[ibuf], ga, sem),
        pltpu.async_copy(acc_b.at[ibuf], gb, sem),
        pltpu.async_copy(t2a.at[ibuf], ta, sem),
        pltpu.async_copy(t2b.at[ibuf], tb, sem),
        pltpu.async_copy(dis.at[ibuf], db, sem),
    ]
    for d in descs:
        d.wait()
    b0 = bb[pl.ds(0, 16)]
    b1 = bb[pl.ds(16, 16)]
    b2 = bb[pl.ds(32, 16)]
    b3 = bb[pl.ds(48, 16)]

    def row(g, _):
        sv = db[pl.ds(g * 16, 16)]
        zero = jnp.zeros((16,), jnp.float32)
        for k in range(16):
            sc = sv[k]
            e = g * 16 + k
            ta0, ta1 = plsc.unpack(ta[e, :],
                                   format=plsc.PackFormat.INTERLEAVED,
                                   preferred_element_type=jnp.float32)
            tb0, tb1 = plsc.unpack(tb[e, :],
                                   format=plsc.PackFormat.INTERLEAVED,
                                   preferred_element_type=jnp.float32)
            zbuf[e, pl.ds(0, 16)] = jnp.maximum(
                (ga[e, pl.ds(0, 16)] + ta0) * sc + b0, zero)
            zbuf[e, pl.ds(16, 16)] = jnp.maximum(
                (ga[e, pl.ds(16, 16)] + ta1) * sc + b1, zero)
            zbuf[e, pl.ds(32, 16)] = jnp.maximum(
                (gb[e, pl.ds(0, 16)] + tb0) * sc + b2, zero)
            zbuf[e, pl.ds(48, 16)] = jnp.maximum(
                (gb[e, pl.ds(16, 16)] + tb1) * sc + b3, zero)
        return 0

    lax.fori_loop(0, 8, row, 0)
    pltpu.sync_copy(zbuf, z_out.at[pl.ds(w * 128, 128)])


_gath_call = pl.kernel(
    _gath_body,
    out_type=jax.ShapeDtypeStruct((BATCH, 2 * HALF), jnp.float32),
    mesh=_MESH,
    scratch_types=[
        pltpu.VMEM((128,), jnp.int32),
        pltpu.VMEM((128, HALF), jnp.float32),
        pltpu.VMEM((128, HALF), jnp.float32),
        pltpu.VMEM((128, HALF), jnp.bfloat16),
        pltpu.VMEM((128, HALF), jnp.bfloat16),
        pltpu.VMEM((128,), jnp.float32),
        pltpu.VMEM((64,), jnp.float32),
        pltpu.VMEM((128, 2 * HALF), jnp.float32),
        pltpu.SemaphoreType.DMA,
    ],
    compiler_params=pltpu.CompilerParams(use_tc_tiling_on_sc=False, needs_layout_passes=False),
)


def _enc_block(x_ref, degp0_ref, degp1_ref, wenc_ref, benc_ref, wc1_ref,
               t1a_ref, t1b_ref, dis_ref, tb1a_ref, tb1b_ref):
    h0 = jnp.dot(x_ref[...], wenc_ref[...],
                 preferred_element_type=jnp.float32) + benc_ref[...][None, :]
    lin1 = jnp.dot(h0, wc1_ref[...], preferred_element_type=jnp.float32)
    deg = degp0_ref[...] + degp1_ref[...] + 1.0
    dis = lax.rsqrt(deg)
    t1 = lin1 * dis[:, None]
    ta = t1[:, :HALF]
    tb = t1[:, HALF:]
    t1a_ref[...] = ta
    t1b_ref[...] = tb
    dis_ref[...] = dis
    tb1a_ref[...] = _shuffle_bf16(ta)
    tb1b_ref[...] = _shuffle_bf16(tb)


def _enc_call(x, degp0, degp1, W_enc, b_enc, W_c1):
    n_blk = N_BLK
    return pl.pallas_call(
        _enc_block,
        grid=(n_blk,),
        in_specs=[
            pl.BlockSpec((R_BLK, 128), lambda r: (r, 0)),
            pl.BlockSpec((R_BLK,), lambda r: (r,)),
            pl.BlockSpec((R_BLK,), lambda r: (r,)),
            pl.BlockSpec((128, 64), lambda r: (0, 0)),
            pl.BlockSpec((64,), lambda r: (0,)),
            pl.BlockSpec((64, 64), lambda r: (0, 0)),
        ],
        out_specs=[
            pl.BlockSpec((R_BLK, HALF), lambda r: (r, 0)),
            pl.BlockSpec((R_BLK, HALF), lambda r: (r, 0)),
            pl.BlockSpec((R_BLK,), lambda r: (r,)),
            pl.BlockSpec((R_BLK, HALF), lambda r: (r, 0)),
            pl.BlockSpec((R_BLK, HALF), lambda r: (r, 0)),
        ],
        out_shape=[
            jax.ShapeDtypeStruct((N_NODES, HALF), jnp.float32),
            jax.ShapeDtypeStruct((N_NODES, HALF), jnp.float32),
            jax.ShapeDtypeStruct((N_NODES,), jnp.float32),
            jax.ShapeDtypeStruct((N_NODES, HALF), jnp.bfloat16),
            jax.ShapeDtypeStruct((N_NODES, HALF), jnp.bfloat16),
        ],
    )(x, degp0, degp1, W_enc, b_enc, W_c1)


def _mid_block(acc_a_ref, acc_b_ref, t1a_ref, t1b_ref, dis_ref, bc1_ref,
               wc2_ref, t2a_ref, t2b_ref):
    dis = dis_ref[...]
    b = bc1_ref[...]
    w = wc2_ref[...]
    ua = (acc_a_ref[...] + t1a_ref[...]) * dis[:, None] + b[None, :HALF]
    ub = (acc_b_ref[...] + t1b_ref[...]) * dis[:, None] + b[None, HALF:]
    h1a = jnp.maximum(ua, 0.0)
    h1b = jnp.maximum(ub, 0.0)
    lin2 = (jnp.dot(h1a, w[:HALF, :], preferred_element_type=jnp.float32)
            + jnp.dot(h1b, w[HALF:, :], preferred_element_type=jnp.float32))
    t2 = lin2 * dis[:, None]
    t2a_ref[...] = _shuffle_bf16(t2[:, :HALF])
    t2b_ref[...] = _shuffle_bf16(t2[:, HALF:])


def _mid_call(acc1a, acc1b, t1a, t1b, dis, b_c1, W_c2):
    n_blk = N_BLK
    return pl.pallas_call(
        _mid_block,
        grid=(n_blk,),
        in_specs=[
            pl.BlockSpec((R_BLK, HALF), lambda r: (r, 0)),
            pl.BlockSpec((R_BLK, HALF), lambda r: (r, 0)),
            pl.BlockSpec((R_BLK, HALF), lambda r: (r, 0)),
            pl.BlockSpec((R_BLK, HALF), lambda r: (r, 0)),
            pl.BlockSpec((R_BLK,), lambda r: (r,)),
            pl.BlockSpec((64,), lambda r: (0,)),
            pl.BlockSpec((64, 64), lambda r: (0, 0)),
        ],
        out_specs=[
            pl.BlockSpec((R_BLK, HALF), lambda r: (r, 0)),
            pl.BlockSpec((R_BLK, HALF), lambda r: (r, 0)),
        ],
        out_shape=[
            jax.ShapeDtypeStruct((N_NODES, HALF), jnp.bfloat16),
            jax.ShapeDtypeStruct((N_NODES, HALF), jnp.bfloat16),
        ],
    )(acc1a, acc1b, t1a, t1b, dis, b_c1, W_c2)


def _out_block(z_ref, wout_ref, bout_ref, o_ref):
    o_ref[...] = jnp.dot(z_ref[...], wout_ref[...],
                         preferred_element_type=jnp.float32) + bout_ref[...][None, :]


def _out_call(z, W_out, b_out):
    return pl.pallas_call(
        _out_block,
        grid=(4,),
        in_specs=[
            pl.BlockSpec((BATCH // 4, 64), lambda r: (r, 0)),
            pl.BlockSpec((64, 128), lambda r: (0, 0)),
            pl.BlockSpec((128,), lambda r: (0,)),
        ],
        out_specs=pl.BlockSpec((BATCH // 4, 128), lambda r: (r, 0)),
        out_shape=jax.ShapeDtypeStruct((BATCH, 128), jnp.float32),
    )(z, W_out, b_out)


@jax.jit
def kernel(x, edge_index, edge_attr, drug_indices,
           W_enc, b_enc, W_c1, b_c1, W_c2, b_c2, W_out, b_out):
    src = edge_index[0].astype(jnp.int32)
    dst = edge_index[1].astype(jnp.int32)
    ew = edge_attr.astype(jnp.float32)
    pad = E_PAD - src.shape[0]
    fill = (jnp.arange(pad, dtype=jnp.int32) * 67) % N_NODES
    src2d = jnp.concatenate([src, fill]).reshape(E_ROWS, 128)
    dst2d = jnp.concatenate([dst, fill]).reshape(E_ROWS, 128)
    ew1d = jnp.concatenate([ew, jnp.zeros((pad,), jnp.float32)])
    drug2d = drug_indices.astype(jnp.int32).reshape(32, 128)

    degp0, degp1 = _deg_call(dst2d, ew1d)
    t1a, t1b, dis, tb1a, tb1b = _enc_call(x, degp0, degp1, W_enc, b_enc, W_c1)
    acc1a, acc1b = _spmm_call(tb1a, tb1b, src2d, dst2d, ew1d)
    tb2a, tb2b = _mid_call(acc1a, acc1b, t1a, t1b, dis, b_c1, W_c2)
    acc2a, acc2b = _spmm_call(tb2a, tb2b, src2d, dst2d, ew1d)
    z = _gath_call(acc2a, acc2b, tb2a, tb2b, dis, b_c2, drug2d)
    return _out_call(z, W_out, b_out)


# speedup vs baseline: 1.4183x; 1.4183x over previous
import functools

import jax
import jax.numpy as jnp
import numpy as np
from jax import lax
from jax.experimental import pallas as pl
from jax.experimental.pallas import tpu as pltpu
from jax.experimental.pallas import tpu_sc as plsc

N_NODES = 50000
N_PAD = 50048
STRIPE = N_PAD // 16
HALF = 32
NC, NS = 2, 16

E_PAD = 802816
E_ROWS = E_PAD // 128
SP_TROWS = E_ROWS // 16
SP_CH = 2
SP_NBLK = SP_TROWS // SP_CH
DG_WROWS = E_ROWS // 32
DG_BLKS = DG_WROWS // 4

BATCH = 4096
R_BLK = 2048
N_BLK = 25

_MESH = plsc.VectorSubcoreMesh(
    core_axis_name="c", subcore_axis_name="s", num_cores=NC, num_subcores=NS)

def _shuffle_bf16(t):
    r = t.shape[0]
    return (t.reshape(r, 2, 16).transpose(0, 2, 1)
            .reshape(r, HALF).astype(jnp.bfloat16))


def _zero_rows(buf, n_rows, stripe_base, acc):
    def zb(e, _):
        buf[e, pl.ds(0, 16)] = jnp.zeros((16,), jnp.float32)
        buf[e, pl.ds(16, 16)] = jnp.zeros((16,), jnp.float32)
        return 0
    lax.fori_loop(0, n_rows, zb, 0)
    full, rem = STRIPE // n_rows, STRIPE % n_rows
    for k in range(full):
        pltpu.sync_copy(buf.at[pl.ds(0, n_rows)],
                        acc.at[pl.ds(stripe_base + k * n_rows, n_rows)])
    if rem:
        pltpu.sync_copy(buf.at[pl.ds(0, rem)],
                        acc.at[pl.ds(stripe_base + full * n_rows, rem)])


def _zero_1d(buf, n, stripe_base, acc):
    def zb(i, _):
        buf[pl.ds(i * 16, 16)] = jnp.zeros((16,), jnp.float32)
        return 0
    lax.fori_loop(0, n // 16, zb, 0)
    full, rem = STRIPE // n, STRIPE % n
    for k in range(full):
        pltpu.sync_copy(buf.at[pl.ds(0, n)],
                        acc.at[pl.ds(stripe_base + k * n, n)])
    if rem:
        pltpu.sync_copy(buf.at[pl.ds(0, rem)],
                        acc.at[pl.ds(stripe_base + full * n, rem)])


def _deg_body(dst2d, ew1d, out_p0, out_p1, acc, dbuf, ewbuf, zbuf):
    c = lax.axis_index("c")
    s = lax.axis_index("s")
    _zero_1d(zbuf, 512, s * STRIPE, acc)
    plsc.subcore_barrier()
    w = c * NS + s

    def blk(b, _):
        row0 = w * DG_WROWS + b * 4
        pltpu.sync_copy(dst2d.at[pl.ds(row0, 4)], dbuf)
        pltpu.sync_copy(ew1d.at[pl.ds(row0 * 128, 512)], ewbuf)
        for j in range(4):
            pltpu.sync_copy(ewbuf.at[pl.ds(j * 128, 128)],
                            acc.at[dbuf.at[j]], add=True)
        return 0

    lax.fori_loop(0, DG_BLKS, blk, 0)
    plsc.subcore_barrier()

    def drain(out):
        full, rem = STRIPE // 512, STRIPE % 512
        for k in range(full + (1 if rem else 0)):
            n = 512 if k < full else rem
            off = s * STRIPE + k * 512
            pltpu.sync_copy(acc.at[pl.ds(off, n)], zbuf.at[pl.ds(0, n)])
            pltpu.sync_copy(zbuf.at[pl.ds(0, n)], out.at[pl.ds(off, n)])

    @pl.when(c == 0)
    def _():
        drain(out_p0)

    @pl.when(c == 1)
    def _():
        drain(out_p1)


_deg_call = pl.kernel(
    _deg_body,
    out_type=[jax.ShapeDtypeStruct((N_PAD,), jnp.float32),
              jax.ShapeDtypeStruct((N_PAD,), jnp.float32)],
    mesh=_MESH,
    scratch_types=[
        pltpu.VMEM_SHARED((N_PAD,), jnp.float32),
        pltpu.VMEM((4, 128), jnp.int32),
        pltpu.VMEM((512,), jnp.float32),
        pltpu.VMEM((512,), jnp.float32),
    ],
)


def _spmm_body(tab_a, tab_b, src2d, dst2d, ew1d, out_a, out_b,
               acc, sbuf, dbuf, ewbuf, rbuf, rbf, isem, gsem, ssem):
    c = lax.axis_index("c")
    s = lax.axis_index("s")
    _zero_rows(rbuf.at[0], SP_CH * 128, s * STRIPE, acc)
    plsc.subcore_barrier()
    NB = SP_NBLK

    def stage(b, p, q):
        r0 = s * SP_TROWS + b * SP_CH
        pltpu.async_copy(src2d.at[pl.ds(r0, SP_CH)], sbuf.at[p], isem)
        pltpu.async_copy(dst2d.at[pl.ds(r0, SP_CH)], dbuf.at[q], isem)
        pltpu.async_copy(ew1d.at[pl.ds(r0 * 128, SP_CH * 128)],
                         ewbuf.at[p], isem)

    def wait_stage(p, q):
        pltpu.make_async_copy(src2d.at[pl.ds(0, SP_CH)], sbuf.at[p],
                              isem).wait()
        pltpu.make_async_copy(dst2d.at[pl.ds(0, SP_CH)], dbuf.at[q],
                              isem).wait()
        pltpu.make_async_copy(ew1d.at[pl.ds(0, SP_CH * 128)], ewbuf.at[p],
                              isem).wait()

    def run(tab, out):
        def fire(p):
            for j in range(SP_CH):
                pltpu.async_copy(tab.at[sbuf.at[p, j]],
                                 rbf.at[p, pl.ds(j * 128, 128)], gsem)

        def wait_fire(p):
            for j in range(SP_CH):
                pltpu.make_async_copy(tab.at[sbuf.at[p, j]],
                                      rbf.at[p, pl.ds(j * 128, 128)],
                                      gsem).wait()

        def scale(p):
            mask = jnp.uint32(0xFFFF0000)

            def body(g, _):
                wv = ewbuf[p, pl.ds(g * 16, 16)]
                for k in range(16):
                    w = wv[k]
                    e = g * 16 + k
                    xi = plsc.bitcast(rbf[p, e, :], jnp.uint32)
                    lo = plsc.bitcast(xi << 16, jnp.float32)
                    hi = plsc.bitcast(xi & mask, jnp.float32)
                    rbuf[p, e, pl.ds(0, 16)] = lo * w
                    rbuf[p, e, pl.ds(16, 16)] = hi * w
                return 0
            lax.fori_loop(0, SP_CH * 8, body, 0)

        def scat(p, q):
            for j in range(SP_CH):
                pltpu.async_copy(rbuf.at[p, pl.ds(j * 128, 128)],
                                 acc.at[dbuf.at[q, j]], ssem, add=True)

        def wait_scat(p, q):
            for j in range(SP_CH):
                pltpu.make_async_copy(rbuf.at[p, pl.ds(j * 128, 128)],
                                      acc.at[dbuf.at[q, j]], ssem).wait()

        stage(0, 0, 0)
        wait_stage(0, 0)
        fire(0)
        stage(1, 1, 1)

        def outer(bb, _):
            for q in range(4):
                b = bb * 4 + q
                p = q % 2

                @pl.when(b + 1 < NB)
                def _():
                    wait_stage(p ^ 1, (q + 1) % 4)

                @pl.when(b > 0)
                def _():
                    wait_scat(p ^ 1, (q - 1) % 4)

                @pl.when(b + 1 < NB)
                def _():
                    fire(p ^ 1)
                wait_fire(p)
                scale(p)
                scat(p, q)

                @pl.when(b + 2 < NB)
                def _():
                    stage(b + 2, p, (q + 2) % 4)
            return 0

        lax.fori_loop(0, NB // 4, outer, 0)
        wait_scat(1, 3)
        plsc.subcore_barrier()
        nbuf = SP_CH * 128
        full, rem = STRIPE // nbuf, STRIPE % nbuf
        for k in range(full + (1 if rem else 0)):
            n = nbuf if k < full else rem
            off = s * STRIPE + k * nbuf
            pltpu.sync_copy(acc.at[pl.ds(off, n)], rbuf.at[0, pl.ds(0, n)])
            pltpu.sync_copy(rbuf.at[0, pl.ds(0, n)], out.at[pl.ds(off, n)])

    @pl.when(c == 0)
    def _():
        run(tab_a, out_a)

    @pl.when(c == 1)
    def _():
        run(tab_b, out_b)


_spmm_call = pl.kernel(
    _spmm_body,
    out_type=[jax.ShapeDtypeStruct((N_PAD, HALF), jnp.float32),
              jax.ShapeDtypeStruct((N_PAD, HALF), jnp.float32)],
    mesh=_MESH,
    scratch_types=[
        pltpu.VMEM_SHARED((N_PAD, HALF), jnp.float32),
        pltpu.VMEM((2, SP_CH, 128), jnp.int32),
        pltpu.VMEM((4, SP_CH, 128), jnp.int32),
        pltpu.VMEM((2, SP_CH * 128), jnp.float32),
        pltpu.VMEM((2, SP_CH * 128, HALF), jnp.float32),
        pltpu.VMEM((2, SP_CH * 128, HALF), jnp.bfloat16),
        pltpu.SemaphoreType.DMA,
        pltpu.SemaphoreType.DMA,
        pltpu.SemaphoreType.DMA,
    ],
    compiler_params=pltpu.CompilerParams(use_tc_tiling_on_sc=False, needs_layout_passes=False),
)


def _gath_body(acc_a, acc_b, t2a, t2b, dis, bvec, drug2d, z_out,
               ibuf, ga, gb, ta, tb, db, bb, zbuf, sem):
    c = lax.axis_index("c")
    s = lax.axis_index("s")
    w = c * NS + s
    pltpu.sync_copy(drug2d.at[w], ibuf)
    pltpu.sync_copy(bvec, bb)
    descs = [
        pltpu.async_copy(acc_a.at[ibuf], ga, sem),
        pltpu.async_copy(acc_b.at[ibuf], gb, sem),
        pltpu.async_copy(t2a.at[ibuf], ta, sem),
        pltpu.async_copy(t2b.at[ibuf], tb, sem),
        pltpu.async_copy(dis.at[ibuf], db, sem),
    ]
    for d in descs:
        d.wait()
    b0 = bb[pl.ds(0, 16)]
    b1 = bb[pl.ds(16, 16)]
    b2 = bb[pl.ds(32, 16)]
    b3 = bb[pl.ds(48, 16)]

    def row(g, _):
        sv = db[pl.ds(g * 16, 16)]
        zero = jnp.zeros((16,), jnp.float32)
        for k in range(16):
            sc = sv[k]
            e = g * 16 + k
            mask = jnp.uint32(0xFFFF0000)
            xa = plsc.bitcast(ta[e, :], jnp.uint32)
            xb = plsc.bitcast(tb[e, :], jnp.uint32)
            ta0 = plsc.bitcast(xa << 16, jnp.float32)
            ta1 = plsc.bitcast(xa & mask, jnp.float32)
            tb0 = plsc.bitcast(xb << 16, jnp.float32)
            tb1 = plsc.bitcast(xb & mask, jnp.float32)
            zbuf[e, pl.ds(0, 16)] = jnp.maximum(
                (ga[e, pl.ds(0, 16)] + ta0) * sc + b0, zero)
            zbuf[e, pl.ds(16, 16)] = jnp.maximum(
                (ga[e, pl.ds(16, 16)] + ta1) * sc + b1, zero)
            zbuf[e, pl.ds(32, 16)] = jnp.maximum(
                (gb[e, pl.ds(0, 16)] + tb0) * sc + b2, zero)
            zbuf[e, pl.ds(48, 16)] = jnp.maximum(
                (gb[e, pl.ds(16, 16)] + tb1) * sc + b3, zero)
        return 0

    lax.fori_loop(0, 8, row, 0)
    pltpu.sync_copy(zbuf, z_out.at[pl.ds(w * 128, 128)])


_gath_call = pl.kernel(
    _gath_body,
    out_type=jax.ShapeDtypeStruct((BATCH, 2 * HALF), jnp.float32),
    mesh=_MESH,
    scratch_types=[
        pltpu.VMEM((128,), jnp.int32),
        pltpu.VMEM((128, HALF), jnp.float32),
        pltpu.VMEM((128, HALF), jnp.float32),
        pltpu.VMEM((128, HALF), jnp.bfloat16),
        pltpu.VMEM((128, HALF), jnp.bfloat16),
        pltpu.VMEM((128,), jnp.float32),
        pltpu.VMEM((64,), jnp.float32),
        pltpu.VMEM((128, 2 * HALF), jnp.float32),
        pltpu.SemaphoreType.DMA,
    ],
    compiler_params=pltpu.CompilerParams(use_tc_tiling_on_sc=False, needs_layout_passes=False),
)


def _enc_block(x_ref, degp0_ref, degp1_ref, wenc_ref, benc_ref, wc1_ref,
               t1a_ref, t1b_ref, dis_ref):
    h0 = jnp.dot(x_ref[...], wenc_ref[...],
                 preferred_element_type=jnp.float32) + benc_ref[...][None, :]
    lin1 = jnp.dot(h0, wc1_ref[...], preferred_element_type=jnp.float32)
    deg = degp0_ref[...] + degp1_ref[...] + 1.0
    dis = lax.rsqrt(deg)
    t1 = lin1 * dis[:, None]
    t1a_ref[...] = t1[:, :HALF]
    t1b_ref[...] = t1[:, HALF:]
    dis_ref[...] = dis


def _enc_call(x, degp0, degp1, W_enc, b_enc, W_c1):
    n_blk = N_BLK
    return pl.pallas_call(
        _enc_block,
        grid=(n_blk,),
        in_specs=[
            pl.BlockSpec((R_BLK, 128), lambda r: (r, 0)),
            pl.BlockSpec((R_BLK,), lambda r: (r,)),
            pl.BlockSpec((R_BLK,), lambda r: (r,)),
            pl.BlockSpec((128, 64), lambda r: (0, 0)),
            pl.BlockSpec((64,), lambda r: (0,)),
            pl.BlockSpec((64, 64), lambda r: (0, 0)),
        ],
        out_specs=[
            pl.BlockSpec((R_BLK, HALF), lambda r: (r, 0)),
            pl.BlockSpec((R_BLK, HALF), lambda r: (r, 0)),
            pl.BlockSpec((R_BLK,), lambda r: (r,)),
        ],
        out_shape=[
            jax.ShapeDtypeStruct((N_NODES, HALF), jnp.float32),
            jax.ShapeDtypeStruct((N_NODES, HALF), jnp.float32),
            jax.ShapeDtypeStruct((N_NODES,), jnp.float32),
        ],
    )(x, degp0, degp1, W_enc, b_enc, W_c1)


def _mid_block(acc_a_ref, acc_b_ref, t1a_ref, t1b_ref, dis_ref, bc1_ref,
               wc2_ref, t2a_ref, t2b_ref):
    dis = dis_ref[...]
    b = bc1_ref[...]
    w = wc2_ref[...]
    ua = (acc_a_ref[...] + t1a_ref[...]) * dis[:, None] + b[None, :HALF]
    ub = (acc_b_ref[...] + t1b_ref[...]) * dis[:, None] + b[None, HALF:]
    h1a = jnp.maximum(ua, 0.0)
    h1b = jnp.maximum(ub, 0.0)
    lin2 = (jnp.dot(h1a, w[:HALF, :], preferred_element_type=jnp.float32)
            + jnp.dot(h1b, w[HALF:, :], preferred_element_type=jnp.float32))
    t2 = lin2 * dis[:, None]
    t2a_ref[...] = t2[:, :HALF]
    t2b_ref[...] = t2[:, HALF:]


def _mid_call(acc1a, acc1b, t1a, t1b, dis, b_c1, W_c2):
    n_blk = N_BLK
    return pl.pallas_call(
        _mid_block,
        grid=(n_blk,),
        in_specs=[
            pl.BlockSpec((R_BLK, HALF), lambda r: (r, 0)),
            pl.BlockSpec((R_BLK, HALF), lambda r: (r, 0)),
            pl.BlockSpec((R_BLK, HALF), lambda r: (r, 0)),
            pl.BlockSpec((R_BLK, HALF), lambda r: (r, 0)),
            pl.BlockSpec((R_BLK,), lambda r: (r,)),
            pl.BlockSpec((64,), lambda r: (0,)),
            pl.BlockSpec((64, 64), lambda r: (0, 0)),
        ],
        out_specs=[
            pl.BlockSpec((R_BLK, HALF), lambda r: (r, 0)),
            pl.BlockSpec((R_BLK, HALF), lambda r: (r, 0)),
        ],
        out_shape=[
            jax.ShapeDtypeStruct((N_NODES, HALF), jnp.float32),
            jax.ShapeDtypeStruct((N_NODES, HALF), jnp.float32),
        ],
    )(acc1a, acc1b, t1a, t1b, dis, b_c1, W_c2)


def _out_block(z_ref, wout_ref, bout_ref, o_ref):
    o_ref[...] = jnp.dot(z_ref[...], wout_ref[...],
                         preferred_element_type=jnp.float32) + bout_ref[...][None, :]


def _out_call(z, W_out, b_out):
    return pl.pallas_call(
        _out_block,
        grid=(4,),
        in_specs=[
            pl.BlockSpec((BATCH // 4, 64), lambda r: (r, 0)),
            pl.BlockSpec((64, 128), lambda r: (0, 0)),
            pl.BlockSpec((128,), lambda r: (0,)),
        ],
        out_specs=pl.BlockSpec((BATCH // 4, 128), lambda r: (r, 0)),
        out_shape=jax.ShapeDtypeStruct((BATCH, 128), jnp.float32),
    )(z, W_out, b_out)


@jax.jit
def kernel(x, edge_index, edge_attr, drug_indices,
           W_enc, b_enc, W_c1, b_c1, W_c2, b_c2, W_out, b_out):
    src = edge_index[0].astype(jnp.int32)
    dst = edge_index[1].astype(jnp.int32)
    ew = edge_attr.astype(jnp.float32)
    pad = E_PAD - src.shape[0]
    fill = (jnp.arange(pad, dtype=jnp.int32) * 67) % N_NODES
    src2d = jnp.concatenate([src, fill]).reshape(E_ROWS, 128)
    dst2d = jnp.concatenate([dst, fill]).reshape(E_ROWS, 128)
    ew1d = jnp.concatenate([ew, jnp.zeros((pad,), jnp.float32)])
    drug2d = drug_indices.astype(jnp.int32).reshape(32, 128)

    degp0, degp1 = _deg_call(dst2d, ew1d)
    t1a, t1b, dis = _enc_call(x, degp0, degp1, W_enc, b_enc, W_c1)
    acc1a, acc1b = _spmm_call(_shuffle_bf16(t1a), _shuffle_bf16(t1b),
                              src2d, dst2d, ew1d)
    t2a, t2b = _mid_call(acc1a, acc1b, t1a, t1b, dis, b_c1, W_c2)
    tb2a, tb2b = _shuffle_bf16(t2a), _shuffle_bf16(t2b)
    acc2a, acc2b = _spmm_call(tb2a, tb2b, src2d, dst2d, ew1d)
    z = _gath_call(acc2a, acc2b, tb2a, tb2b, dis, b_c2, drug2d)
    return _out_call(z, W_out, b_out)


# speedup vs baseline: 2.5476x; 1.7963x over previous
import functools

import jax
import jax.numpy as jnp
import numpy as np
from jax import lax
from jax.experimental import pallas as pl
from jax.experimental.pallas import tpu as pltpu
from jax.experimental.pallas import tpu_sc as plsc

N_NODES = 50000
N_PAD = 50048
STRIPE = N_PAD // 16
HALF = 32
NC, NS = 2, 16

E_PAD = 802816
E_ROWS = E_PAD // 128
SP_TROWS = E_ROWS // 16
SP_CH = 2
SP_NBLK = SP_TROWS // SP_CH
DG_WROWS = E_ROWS // 32
DG_BLKS = DG_WROWS // 4

BATCH = 4096
R_BLK = 2048
N_BLK = 25

_MESH = plsc.VectorSubcoreMesh(
    core_axis_name="c", subcore_axis_name="s", num_cores=NC, num_subcores=NS)

def _shuffle_bf16(t):
    r = t.shape[0]
    return (t.reshape(r, 2, 16).transpose(0, 2, 1)
            .reshape(r, HALF).astype(jnp.bfloat16))


def _zero_rows(buf, n_rows, stripe_base, acc):
    def zb(e, _):
        buf[e, pl.ds(0, 16)] = jnp.zeros((16,), jnp.float32)
        buf[e, pl.ds(16, 16)] = jnp.zeros((16,), jnp.float32)
        return 0
    lax.fori_loop(0, n_rows, zb, 0)
    full, rem = STRIPE // n_rows, STRIPE % n_rows
    for k in range(full):
        pltpu.sync_copy(buf.at[pl.ds(0, n_rows)],
                        acc.at[pl.ds(stripe_base + k * n_rows, n_rows)])
    if rem:
        pltpu.sync_copy(buf.at[pl.ds(0, rem)],
                        acc.at[pl.ds(stripe_base + full * n_rows, rem)])


def _zero_1d(buf, n, stripe_base, acc):
    def zb(i, _):
        buf[pl.ds(i * 16, 16)] = jnp.zeros((16,), jnp.float32)
        return 0
    lax.fori_loop(0, n // 16, zb, 0)
    full, rem = STRIPE // n, STRIPE % n
    for k in range(full):
        pltpu.sync_copy(buf.at[pl.ds(0, n)],
                        acc.at[pl.ds(stripe_base + k * n, n)])
    if rem:
        pltpu.sync_copy(buf.at[pl.ds(0, rem)],
                        acc.at[pl.ds(stripe_base + full * n, rem)])


def _deg_body(dst2d, ew1d, out_p0, out_p1, acc, dbuf, ewbuf, zbuf):
    c = lax.axis_index("c")
    s = lax.axis_index("s")
    _zero_1d(zbuf, 512, s * STRIPE, acc)
    plsc.subcore_barrier()
    w = c * NS + s

    def blk(b, _):
        row0 = w * DG_WROWS + b * 4
        pltpu.sync_copy(dst2d.at[pl.ds(row0, 4)], dbuf)
        pltpu.sync_copy(ew1d.at[pl.ds(row0 * 128, 512)], ewbuf)
        for j in range(4):
            pltpu.sync_copy(ewbuf.at[pl.ds(j * 128, 128)],
                            acc.at[dbuf.at[j]], add=True)
        return 0

    lax.fori_loop(0, DG_BLKS, blk, 0)
    plsc.subcore_barrier()

    def drain(out):
        full, rem = STRIPE // 512, STRIPE % 512
        for k in range(full + (1 if rem else 0)):
            n = 512 if k < full else rem
            off = s * STRIPE + k * 512
            pltpu.sync_copy(acc.at[pl.ds(off, n)], zbuf.at[pl.ds(0, n)])
            pltpu.sync_copy(zbuf.at[pl.ds(0, n)], out.at[pl.ds(off, n)])

    @pl.when(c == 0)
    def _():
        drain(out_p0)

    @pl.when(c == 1)
    def _():
        drain(out_p1)


_deg_call = pl.kernel(
    _deg_body,
    out_type=[jax.ShapeDtypeStruct((N_PAD,), jnp.float32),
              jax.ShapeDtypeStruct((N_PAD,), jnp.float32)],
    mesh=_MESH,
    scratch_types=[
        pltpu.VMEM_SHARED((N_PAD,), jnp.float32),
        pltpu.VMEM((4, 128), jnp.int32),
        pltpu.VMEM((512,), jnp.float32),
        pltpu.VMEM((512,), jnp.float32),
    ],
)


def _spmm_body(tab_a, tab_b, src2d, dst2d, ew1d, out_a, out_b,
               acc, sbuf, dbuf, ewbuf, rbuf, isem, gsem, ssem):
    c = lax.axis_index("c")
    s = lax.axis_index("s")
    _zero_rows(rbuf.at[0], SP_CH * 128, s * STRIPE, acc)
    plsc.subcore_barrier()
    NB = SP_NBLK

    def stage(b, p, q):
        r0 = s * SP_TROWS + b * SP_CH
        pltpu.async_copy(src2d.at[pl.ds(r0, SP_CH)], sbuf.at[p], isem)
        pltpu.async_copy(dst2d.at[pl.ds(r0, SP_CH)], dbuf.at[q], isem)
        pltpu.async_copy(ew1d.at[pl.ds(r0 * 128, SP_CH * 128)],
                         ewbuf.at[p], isem)

    def wait_stage(p, q):
        pltpu.make_async_copy(src2d.at[pl.ds(0, SP_CH)], sbuf.at[p],
                              isem).wait()
        pltpu.make_async_copy(dst2d.at[pl.ds(0, SP_CH)], dbuf.at[q],
                              isem).wait()
        pltpu.make_async_copy(ew1d.at[pl.ds(0, SP_CH * 128)], ewbuf.at[p],
                              isem).wait()

    def run(tab, out):
        def fire(p):
            for j in range(SP_CH):
                pltpu.async_copy(tab.at[sbuf.at[p, j]],
                                 rbuf.at[p, pl.ds(j * 128, 128)], gsem)

        def wait_fire(p):
            for j in range(SP_CH):
                pltpu.make_async_copy(tab.at[sbuf.at[p, j]],
                                      rbuf.at[p, pl.ds(j * 128, 128)],
                                      gsem).wait()

        def scale(p):
            def body(g, _):
                wv = ewbuf[p, pl.ds(g * 16, 16)]
                for k in range(16):
                    w = wv[k]
                    e = g * 16 + k
                    rbuf[p, e, pl.ds(0, 16)] = rbuf[p, e, pl.ds(0, 16)] * w
                    rbuf[p, e, pl.ds(16, 16)] = rbuf[p, e, pl.ds(16, 16)] * w
                return 0
            lax.fori_loop(0, SP_CH * 8, body, 0)

        def scat(p, q):
            for j in range(SP_CH):
                pltpu.async_copy(rbuf.at[p, pl.ds(j * 128, 128)],
                                 acc.at[dbuf.at[q, j]], ssem, add=True)

        def wait_scat(p, q):
            for j in range(SP_CH):
                pltpu.make_async_copy(rbuf.at[p, pl.ds(j * 128, 128)],
                                      acc.at[dbuf.at[q, j]], ssem).wait()

        stage(0, 0, 0)
        wait_stage(0, 0)
        fire(0)
        stage(1, 1, 1)

        def outer(bb, _):
            for q in range(4):
                b = bb * 4 + q
                p = q % 2

                @pl.when(b + 1 < NB)
                def _():
                    wait_stage(p ^ 1, (q + 1) % 4)

                @pl.when(b > 0)
                def _():
                    wait_scat(p ^ 1, (q - 1) % 4)

                @pl.when(b + 1 < NB)
                def _():
                    fire(p ^ 1)
                wait_fire(p)
                scale(p)
                scat(p, q)

                @pl.when(b + 2 < NB)
                def _():
                    stage(b + 2, p, (q + 2) % 4)
            return 0

        lax.fori_loop(0, NB // 4, outer, 0)
        wait_scat(1, 3)
        plsc.subcore_barrier()
        nbuf = SP_CH * 128
        full, rem = STRIPE // nbuf, STRIPE % nbuf
        for k in range(full + (1 if rem else 0)):
            n = nbuf if k < full else rem
            off = s * STRIPE + k * nbuf
            pltpu.sync_copy(acc.at[pl.ds(off, n)], rbuf.at[0, pl.ds(0, n)])
            pltpu.sync_copy(rbuf.at[0, pl.ds(0, n)], out.at[pl.ds(off, n)])

    @pl.when(c == 0)
    def _():
        run(tab_a, out_a)

    @pl.when(c == 1)
    def _():
        run(tab_b, out_b)


_spmm_call = pl.kernel(
    _spmm_body,
    out_type=[jax.ShapeDtypeStruct((N_PAD, HALF), jnp.float32),
              jax.ShapeDtypeStruct((N_PAD, HALF), jnp.float32)],
    mesh=_MESH,
    scratch_types=[
        pltpu.VMEM_SHARED((N_PAD, HALF), jnp.float32),
        pltpu.VMEM((2, SP_CH, 128), jnp.int32),
        pltpu.VMEM((4, SP_CH, 128), jnp.int32),
        pltpu.VMEM((2, SP_CH * 128), jnp.float32),
        pltpu.VMEM((2, SP_CH * 128, HALF), jnp.float32),
        pltpu.SemaphoreType.DMA,
        pltpu.SemaphoreType.DMA,
        pltpu.SemaphoreType.DMA,
    ],
    compiler_params=pltpu.CompilerParams(use_tc_tiling_on_sc=False),
)


def _gath_body(acc_a, acc_b, t2a, t2b, dis, bvec, drug2d, z_out,
               ibuf, ga, gb, ta, tb, db, bb, zbuf, sem):
    c = lax.axis_index("c")
    s = lax.axis_index("s")
    w = c * NS + s
    pltpu.sync_copy(drug2d.at[w], ibuf)
    pltpu.sync_copy(bvec, bb)
    descs = [
        pltpu.async_copy(acc_a.at[ibuf], ga, sem),
        pltpu.async_copy(acc_b.at[ibuf], gb, sem),
        pltpu.async_copy(t2a.at[ibuf], ta, sem),
        pltpu.async_copy(t2b.at[ibuf], tb, sem),
        pltpu.async_copy(dis.at[ibuf], db, sem),
    ]
    for d in descs:
        d.wait()
    b0 = bb[pl.ds(0, 16)]
    b1 = bb[pl.ds(16, 16)]
    b2 = bb[pl.ds(32, 16)]
    b3 = bb[pl.ds(48, 16)]

    def row(g, _):
        sv = db[pl.ds(g * 16, 16)]
        zero = jnp.zeros((16,), jnp.float32)
        for k in range(16):
            sc = sv[k]
            e = g * 16 + k
            ta0 = ta[e, pl.ds(0, 16)]
            ta1 = ta[e, pl.ds(16, 16)]
            tb0 = tb[e, pl.ds(0, 16)]
            tb1 = tb[e, pl.ds(16, 16)]
            zbuf[e, pl.ds(0, 16)] = jnp.maximum(
                (ga[e, pl.ds(0, 16)] + ta0) * sc + b0, zero)
            zbuf[e, pl.ds(16, 16)] = jnp.maximum(
                (ga[e, pl.ds(16, 16)] + ta1) * sc + b1, zero)
            zbuf[e, pl.ds(32, 16)] = jnp.maximum(
                (gb[e, pl.ds(0, 16)] + tb0) * sc + b2, zero)
            zbuf[e, pl.ds(48, 16)] = jnp.maximum(
                (gb[e, pl.ds(16, 16)] + tb1) * sc + b3, zero)
        return 0

    lax.fori_loop(0, 8, row, 0)
    pltpu.sync_copy(zbuf, z_out.at[pl.ds(w * 128, 128)])


_gath_call = pl.kernel(
    _gath_body,
    out_type=jax.ShapeDtypeStruct((BATCH, 2 * HALF), jnp.float32),
    mesh=_MESH,
    scratch_types=[
        pltpu.VMEM((128,), jnp.int32),
        pltpu.VMEM((128, HALF), jnp.float32),
        pltpu.VMEM((128, HALF), jnp.float32),
        pltpu.VMEM((128, HALF), jnp.float32),
        pltpu.VMEM((128, HALF), jnp.float32),
        pltpu.VMEM((128,), jnp.float32),
        pltpu.VMEM((64,), jnp.float32),
        pltpu.VMEM((128, 2 * HALF), jnp.float32),
        pltpu.SemaphoreType.DMA,
    ],
    compiler_params=pltpu.CompilerParams(use_tc_tiling_on_sc=False),
)


def _enc_block(x_ref, degp0_ref, degp1_ref, wenc_ref, benc_ref, wc1_ref,
               t1a_ref, t1b_ref, dis_ref):
    h0 = jnp.dot(x_ref[...], wenc_ref[...],
                 preferred_element_type=jnp.float32) + benc_ref[...][None, :]
    lin1 = jnp.dot(h0, wc1_ref[...], preferred_element_type=jnp.float32)
    deg = degp0_ref[...] + degp1_ref[...] + 1.0
    dis = lax.rsqrt(deg)
    t1 = lin1 * dis[:, None]
    t1a_ref[...] = t1[:, :HALF]
    t1b_ref[...] = t1[:, HALF:]
    dis_ref[...] = dis


def _enc_call(x, degp0, degp1, W_enc, b_enc, W_c1):
    n_blk = N_BLK
    return pl.pallas_call(
        _enc_block,
        grid=(n_blk,),
        in_specs=[
            pl.BlockSpec((R_BLK, 128), lambda r: (r, 0)),
            pl.BlockSpec((R_BLK,), lambda r: (r,)),
            pl.BlockSpec((R_BLK,), lambda r: (r,)),
            pl.BlockSpec((128, 64), lambda r: (0, 0)),
            pl.BlockSpec((64,), lambda r: (0,)),
            pl.BlockSpec((64, 64), lambda r: (0, 0)),
        ],
        out_specs=[
            pl.BlockSpec((R_BLK, HALF), lambda r: (r, 0)),
            pl.BlockSpec((R_BLK, HALF), lambda r: (r, 0)),
            pl.BlockSpec((R_BLK,), lambda r: (r,)),
        ],
        out_shape=[
            jax.ShapeDtypeStruct((N_NODES, HALF), jnp.float32),
            jax.ShapeDtypeStruct((N_NODES, HALF), jnp.float32),
            jax.ShapeDtypeStruct((N_NODES,), jnp.float32),
        ],
    )(x, degp0, degp1, W_enc, b_enc, W_c1)


def _mid_block(acc_a_ref, acc_b_ref, t1a_ref, t1b_ref, dis_ref, bc1_ref,
               wc2_ref, t2a_ref, t2b_ref):
    dis = dis_ref[...]
    b = bc1_ref[...]
    w = wc2_ref[...]
    ua = (acc_a_ref[...] + t1a_ref[...]) * dis[:, None] + b[None, :HALF]
    ub = (acc_b_ref[...] + t1b_ref[...]) * dis[:, None] + b[None, HALF:]
    h1a = jnp.maximum(ua, 0.0)
    h1b = jnp.maximum(ub, 0.0)
    lin2 = (jnp.dot(h1a, w[:HALF, :], preferred_element_type=jnp.float32)
            + jnp.dot(h1b, w[HALF:, :], preferred_element_type=jnp.float32))
    t2 = lin2 * dis[:, None]
    t2a_ref[...] = t2[:, :HALF]
    t2b_ref[...] = t2[:, HALF:]


def _mid_call(acc1a, acc1b, t1a, t1b, dis, b_c1, W_c2):
    n_blk = N_BLK
    return pl.pallas_call(
        _mid_block,
        grid=(n_blk,),
        in_specs=[
            pl.BlockSpec((R_BLK, HALF), lambda r: (r, 0)),
            pl.BlockSpec((R_BLK, HALF), lambda r: (r, 0)),
            pl.BlockSpec((R_BLK, HALF), lambda r: (r, 0)),
            pl.BlockSpec((R_BLK, HALF), lambda r: (r, 0)),
            pl.BlockSpec((R_BLK,), lambda r: (r,)),
            pl.BlockSpec((64,), lambda r: (0,)),
            pl.BlockSpec((64, 64), lambda r: (0, 0)),
        ],
        out_specs=[
            pl.BlockSpec((R_BLK, HALF), lambda r: (r, 0)),
            pl.BlockSpec((R_BLK, HALF), lambda r: (r, 0)),
        ],
        out_shape=[
            jax.ShapeDtypeStruct((N_NODES, HALF), jnp.float32),
            jax.ShapeDtypeStruct((N_NODES, HALF), jnp.float32),
        ],
    )(acc1a, acc1b, t1a, t1b, dis, b_c1, W_c2)


def _out_block(z_ref, wout_ref, bout_ref, o_ref):
    o_ref[...] = jnp.dot(z_ref[...], wout_ref[...],
                         preferred_element_type=jnp.float32) + bout_ref[...][None, :]


def _out_call(z, W_out, b_out):
    return pl.pallas_call(
        _out_block,
        grid=(4,),
        in_specs=[
            pl.BlockSpec((BATCH // 4, 64), lambda r: (r, 0)),
            pl.BlockSpec((64, 128), lambda r: (0, 0)),
            pl.BlockSpec((128,), lambda r: (0,)),
        ],
        out_specs=pl.BlockSpec((BATCH // 4, 128), lambda r: (r, 0)),
        out_shape=jax.ShapeDtypeStruct((BATCH, 128), jnp.float32),
    )(z, W_out, b_out)


@jax.jit
def kernel(x, edge_index, edge_attr, drug_indices,
           W_enc, b_enc, W_c1, b_c1, W_c2, b_c2, W_out, b_out):
    src = edge_index[0].astype(jnp.int32)
    dst = edge_index[1].astype(jnp.int32)
    ew = edge_attr.astype(jnp.float32)
    pad = E_PAD - src.shape[0]
    fill = (jnp.arange(pad, dtype=jnp.int32) * 67) % N_NODES
    src2d = jnp.concatenate([src, fill]).reshape(E_ROWS, 128)
    dst2d = jnp.concatenate([dst, fill]).reshape(E_ROWS, 128)
    ew1d = jnp.concatenate([ew, jnp.zeros((pad,), jnp.float32)])
    drug2d = drug_indices.astype(jnp.int32).reshape(32, 128)

    degp0, degp1 = _deg_call(dst2d, ew1d)
    t1a, t1b, dis = _enc_call(x, degp0, degp1, W_enc, b_enc, W_c1)
    acc1a, acc1b = _spmm_call(t1a, t1b, src2d, dst2d, ew1d)
    t2a, t2b = _mid_call(acc1a, acc1b, t1a, t1b, dis, b_c1, W_c2)
    acc2a, acc2b = _spmm_call(t2a, t2b, src2d, dst2d, ew1d)
    z = _gath_call(acc2a, acc2b, t2a, t2b, dis, b_c2, drug2d)
    return _out_call(z, W_out, b_out)


# speedup vs baseline: 2.6126x; 1.0255x over previous
import functools

import jax
import jax.numpy as jnp
import numpy as np
from jax import lax
from jax.experimental import pallas as pl
from jax.experimental.pallas import tpu as pltpu
from jax.experimental.pallas import tpu_sc as plsc

N_NODES = 50000
N_PAD = 50048
STRIPE = N_PAD // 16
HALF = 32
NC, NS = 2, 16

E_PAD = 802816
E_ROWS = E_PAD // 128
SP_TROWS = E_ROWS // 16
SP_CH = 2
SP_NBLK = SP_TROWS // SP_CH
DG_WROWS = E_ROWS // 32
DG_CH = 2
DG_NBLK = DG_WROWS // DG_CH

BATCH = 4096
R_BLK = 2048
N_BLK = 25

_MESH = plsc.VectorSubcoreMesh(
    core_axis_name="c", subcore_axis_name="s", num_cores=NC, num_subcores=NS)

def _shuffle_bf16(t):
    r = t.shape[0]
    return (t.reshape(r, 2, 16).transpose(0, 2, 1)
            .reshape(r, HALF).astype(jnp.bfloat16))


def _zero_rows(buf, n_rows, stripe_base, acc):
    def zb(e, _):
        buf[e, pl.ds(0, 16)] = jnp.zeros((16,), jnp.float32)
        buf[e, pl.ds(16, 16)] = jnp.zeros((16,), jnp.float32)
        return 0
    lax.fori_loop(0, n_rows, zb, 0)
    full, rem = STRIPE // n_rows, STRIPE % n_rows
    for k in range(full):
        pltpu.sync_copy(buf.at[pl.ds(0, n_rows)],
                        acc.at[pl.ds(stripe_base + k * n_rows, n_rows)])
    if rem:
        pltpu.sync_copy(buf.at[pl.ds(0, rem)],
                        acc.at[pl.ds(stripe_base + full * n_rows, rem)])


def _zero_1d(buf, n, stripe_base, acc):
    def zb(i, _):
        buf[pl.ds(i * 16, 16)] = jnp.zeros((16,), jnp.float32)
        return 0
    lax.fori_loop(0, n // 16, zb, 0)
    full, rem = STRIPE // n, STRIPE % n
    for k in range(full):
        pltpu.sync_copy(buf.at[pl.ds(0, n)],
                        acc.at[pl.ds(stripe_base + k * n, n)])
    if rem:
        pltpu.sync_copy(buf.at[pl.ds(0, rem)],
                        acc.at[pl.ds(stripe_base + full * n, rem)])


def _deg_body(dst2d, ew1d, out_p0, out_p1, acc, dbuf, ewbuf, zbuf, isem):
    c = lax.axis_index("c")
    s = lax.axis_index("s")
    _zero_1d(zbuf, 512, s * STRIPE, acc)
    plsc.subcore_barrier()
    w = c * NS + s
    NB = DG_NBLK

    def stage(b, p):
        row0 = w * DG_WROWS + b * DG_CH
        pltpu.async_copy(dst2d.at[pl.ds(row0, DG_CH)], dbuf.at[p], isem)
        pltpu.async_copy(ew1d.at[pl.ds(row0 * 128, DG_CH * 128)],
                         ewbuf.at[p], isem)

    def wait_stage(p):
        pltpu.make_async_copy(dst2d.at[pl.ds(0, DG_CH)], dbuf.at[p],
                              isem).wait()
        pltpu.make_async_copy(ew1d.at[pl.ds(0, DG_CH * 128)], ewbuf.at[p],
                              isem).wait()

    stage(0, 0)

    def blk(bb, _):
        for p in range(2):
            b = bb * 2 + p
            wait_stage(p)

            @pl.when(b + 1 < NB)
            def _():
                stage(b + 1, p ^ 1)
            for j in range(DG_CH):
                pltpu.sync_copy(ewbuf.at[p, pl.ds(j * 128, 128)],
                                acc.at[dbuf.at[p, j]], add=True)
        return 0

    lax.fori_loop(0, NB // 2, blk, 0)
    plsc.subcore_barrier()

    def drain(out):
        full, rem = STRIPE // 512, STRIPE % 512
        for k in range(full + (1 if rem else 0)):
            n = 512 if k < full else rem
            off = s * STRIPE + k * 512
            pltpu.sync_copy(acc.at[pl.ds(off, n)], zbuf.at[pl.ds(0, n)])
            pltpu.sync_copy(zbuf.at[pl.ds(0, n)], out.at[pl.ds(off, n)])

    @pl.when(c == 0)
    def _():
        drain(out_p0)

    @pl.when(c == 1)
    def _():
        drain(out_p1)


_deg_call = pl.kernel(
    _deg_body,
    out_type=[jax.ShapeDtypeStruct((N_PAD,), jnp.float32),
              jax.ShapeDtypeStruct((N_PAD,), jnp.float32)],
    mesh=_MESH,
    scratch_types=[
        pltpu.VMEM_SHARED((N_PAD,), jnp.float32),
        pltpu.VMEM((2, DG_CH, 128), jnp.int32),
        pltpu.VMEM((2, DG_CH * 128), jnp.float32),
        pltpu.VMEM((512,), jnp.float32),
        pltpu.SemaphoreType.DMA,
    ],
)


def _spmm_body(tab_a, tab_b, src2d, dst2d, ew1d, out_a, out_b,
               acc, sbuf, dbuf, ewbuf, rbuf, isem, gsem, ssem):
    c = lax.axis_index("c")
    s = lax.axis_index("s")
    _zero_rows(rbuf.at[0], SP_CH * 128, s * STRIPE, acc)
    plsc.subcore_barrier()
    NB = SP_NBLK

    def stage(b, p, q):
        r0 = s * SP_TROWS + b * SP_CH
        pltpu.async_copy(src2d.at[pl.ds(r0, SP_CH)], sbuf.at[p], isem)
        pltpu.async_copy(dst2d.at[pl.ds(r0, SP_CH)], dbuf.at[q], isem)
        pltpu.async_copy(ew1d.at[pl.ds(r0 * 128, SP_CH * 128)],
                         ewbuf.at[p], isem)

    def wait_stage(p, q):
        pltpu.make_async_copy(src2d.at[pl.ds(0, SP_CH)], sbuf.at[p],
                              isem).wait()
        pltpu.make_async_copy(dst2d.at[pl.ds(0, SP_CH)], dbuf.at[q],
                              isem).wait()
        pltpu.make_async_copy(ew1d.at[pl.ds(0, SP_CH * 128)], ewbuf.at[p],
                              isem).wait()

    def run(tab, out):
        def fire(p):
            for j in range(SP_CH):
                pltpu.async_copy(tab.at[sbuf.at[p, j]],
                                 rbuf.at[p, pl.ds(j * 128, 128)], gsem)

        def wait_fire(p):
            for j in range(SP_CH):
                pltpu.make_async_copy(tab.at[sbuf.at[p, j]],
                                      rbuf.at[p, pl.ds(j * 128, 128)],
                                      gsem).wait()

        def scale(p):
            def body(g, _):
                wv = ewbuf[p, pl.ds(g * 16, 16)]
                for k in range(16):
                    w = wv[k]
                    e = g * 16 + k
                    rbuf[p, e, pl.ds(0, 16)] = rbuf[p, e, pl.ds(0, 16)] * w
                    rbuf[p, e, pl.ds(16, 16)] = rbuf[p, e, pl.ds(16, 16)] * w
                return 0
            lax.fori_loop(0, SP_CH * 8, body, 0)

        def scat(p, q):
            for j in range(SP_CH):
                pltpu.async_copy(rbuf.at[p, pl.ds(j * 128, 128)],
                                 acc.at[dbuf.at[q, j]], ssem, add=True)

        def wait_scat(p, q):
            for j in range(SP_CH):
                pltpu.make_async_copy(rbuf.at[p, pl.ds(j * 128, 128)],
                                      acc.at[dbuf.at[q, j]], ssem).wait()

        stage(0, 0, 0)
        wait_stage(0, 0)
        fire(0)
        stage(1, 1, 1)

        def outer(bb, _):
            for q in range(4):
                b = bb * 4 + q
                p = q % 2

                @pl.when(b + 1 < NB)
                def _():
                    wait_stage(p ^ 1, (q + 1) % 4)

                @pl.when(b > 0)
                def _():
                    wait_scat(p ^ 1, (q - 1) % 4)

                @pl.when(b + 1 < NB)
                def _():
                    fire(p ^ 1)
                wait_fire(p)
                scale(p)
                scat(p, q)

                @pl.when(b + 2 < NB)
                def _():
                    stage(b + 2, p, (q + 2) % 4)
            return 0

        lax.fori_loop(0, NB // 4, outer, 0)
        wait_scat(1, 3)
        plsc.subcore_barrier()
        nbuf = SP_CH * 128
        full, rem = STRIPE // nbuf, STRIPE % nbuf
        for k in range(full + (1 if rem else 0)):
            n = nbuf if k < full else rem
            off = s * STRIPE + k * nbuf
            pltpu.sync_copy(acc.at[pl.ds(off, n)], rbuf.at[0, pl.ds(0, n)])
            pltpu.sync_copy(rbuf.at[0, pl.ds(0, n)], out.at[pl.ds(off, n)])

    @pl.when(c == 0)
    def _():
        run(tab_a, out_a)

    @pl.when(c == 1)
    def _():
        run(tab_b, out_b)


_spmm_call = pl.kernel(
    _spmm_body,
    out_type=[jax.ShapeDtypeStruct((N_PAD, HALF), jnp.float32),
              jax.ShapeDtypeStruct((N_PAD, HALF), jnp.float32)],
    mesh=_MESH,
    scratch_types=[
        pltpu.VMEM_SHARED((N_PAD, HALF), jnp.float32),
        pltpu.VMEM((2, SP_CH, 128), jnp.int32),
        pltpu.VMEM((4, SP_CH, 128), jnp.int32),
        pltpu.VMEM((2, SP_CH * 128), jnp.float32),
        pltpu.VMEM((2, SP_CH * 128, HALF), jnp.float32),
        pltpu.SemaphoreType.DMA,
        pltpu.SemaphoreType.DMA,
        pltpu.SemaphoreType.DMA,
    ],
    compiler_params=pltpu.CompilerParams(use_tc_tiling_on_sc=False),
)


def _gath_body(acc_a, acc_b, t2a, t2b, dis, bvec, drug2d, z_out,
               ibuf, ga, gb, ta, tb, db, bb, zbuf, sem):
    c = lax.axis_index("c")
    s = lax.axis_index("s")
    w = c * NS + s
    pltpu.sync_copy(drug2d.at[w], ibuf)
    pltpu.sync_copy(bvec, bb)
    descs = [
        pltpu.async_copy(acc_a.at[ibuf], ga, sem),
        pltpu.async_copy(acc_b.at[ibuf], gb, sem),
        pltpu.async_copy(t2a.at[ibuf], ta, sem),
        pltpu.async_copy(t2b.at[ibuf], tb, sem),
        pltpu.async_copy(dis.at[ibuf], db, sem),
    ]
    for d in descs:
        d.wait()
    b0 = bb[pl.ds(0, 16)]
    b1 = bb[pl.ds(16, 16)]
    b2 = bb[pl.ds(32, 16)]
    b3 = bb[pl.ds(48, 16)]

    def row(g, _):
        sv = db[pl.ds(g * 16, 16)]
        zero = jnp.zeros((16,), jnp.float32)
        for k in range(16):
            sc = sv[k]
            e = g * 16 + k
            ta0 = ta[e, pl.ds(0, 16)]
            ta1 = ta[e, pl.ds(16, 16)]
            tb0 = tb[e, pl.ds(0, 16)]
            tb1 = tb[e, pl.ds(16, 16)]
            zbuf[e, pl.ds(0, 16)] = jnp.maximum(
                (ga[e, pl.ds(0, 16)] + ta0) * sc + b0, zero)
            zbuf[e, pl.ds(16, 16)] = jnp.maximum(
                (ga[e, pl.ds(16, 16)] + ta1) * sc + b1, zero)
            zbuf[e, pl.ds(32, 16)] = jnp.maximum(
                (gb[e, pl.ds(0, 16)] + tb0) * sc + b2, zero)
            zbuf[e, pl.ds(48, 16)] = jnp.maximum(
                (gb[e, pl.ds(16, 16)] + tb1) * sc + b3, zero)
        return 0

    lax.fori_loop(0, 8, row, 0)
    pltpu.sync_copy(zbuf, z_out.at[pl.ds(w * 128, 128)])


_gath_call = pl.kernel(
    _gath_body,
    out_type=jax.ShapeDtypeStruct((BATCH, 2 * HALF), jnp.float32),
    mesh=_MESH,
    scratch_types=[
        pltpu.VMEM((128,), jnp.int32),
        pltpu.VMEM((128, HALF), jnp.float32),
        pltpu.VMEM((128, HALF), jnp.float32),
        pltpu.VMEM((128, HALF), jnp.float32),
        pltpu.VMEM((128, HALF), jnp.float32),
        pltpu.VMEM((128,), jnp.float32),
        pltpu.VMEM((64,), jnp.float32),
        pltpu.VMEM((128, 2 * HALF), jnp.float32),
        pltpu.SemaphoreType.DMA,
    ],
    compiler_params=pltpu.CompilerParams(use_tc_tiling_on_sc=False),
)


def _enc_block(x_ref, degp0_ref, degp1_ref, wenc_ref, benc_ref, wc1_ref,
               t1a_ref, t1b_ref, dis_ref):
    h0 = jnp.dot(x_ref[...], wenc_ref[...],
                 preferred_element_type=jnp.float32) + benc_ref[...][None, :]
    lin1 = jnp.dot(h0, wc1_ref[...], preferred_element_type=jnp.float32)
    deg = degp0_ref[...] + degp1_ref[...] + 1.0
    dis = lax.rsqrt(deg)
    t1 = lin1 * dis[:, None]
    t1a_ref[...] = t1[:, :HALF]
    t1b_ref[...] = t1[:, HALF:]
    dis_ref[...] = dis


def _enc_call(x, degp0, degp1, W_enc, b_enc, W_c1):
    n_blk = N_BLK
    return pl.pallas_call(
        _enc_block,
        grid=(n_blk,),
        in_specs=[
            pl.BlockSpec((R_BLK, 128), lambda r: (r, 0)),
            pl.BlockSpec((R_BLK,), lambda r: (r,)),
            pl.BlockSpec((R_BLK,), lambda r: (r,)),
            pl.BlockSpec((128, 64), lambda r: (0, 0)),
            pl.BlockSpec((64,), lambda r: (0,)),
            pl.BlockSpec((64, 64), lambda r: (0, 0)),
        ],
        out_specs=[
            pl.BlockSpec((R_BLK, HALF), lambda r: (r, 0)),
            pl.BlockSpec((R_BLK, HALF), lambda r: (r, 0)),
            pl.BlockSpec((R_BLK,), lambda r: (r,)),
        ],
        out_shape=[
            jax.ShapeDtypeStruct((N_NODES, HALF), jnp.float32),
            jax.ShapeDtypeStruct((N_NODES, HALF), jnp.float32),
            jax.ShapeDtypeStruct((N_NODES,), jnp.float32),
        ],
    )(x, degp0, degp1, W_enc, b_enc, W_c1)


def _mid_block(acc_a_ref, acc_b_ref, t1a_ref, t1b_ref, dis_ref, bc1_ref,
               wc2_ref, t2a_ref, t2b_ref):
    dis = dis_ref[...]
    b = bc1_ref[...]
    w = wc2_ref[...]
    ua = (acc_a_ref[...] + t1a_ref[...]) * dis[:, None] + b[None, :HALF]
    ub = (acc_b_ref[...] + t1b_ref[...]) * dis[:, None] + b[None, HALF:]
    h1a = jnp.maximum(ua, 0.0)
    h1b = jnp.maximum(ub, 0.0)
    lin2 = (jnp.dot(h1a, w[:HALF, :], preferred_element_type=jnp.float32)
            + jnp.dot(h1b, w[HALF:, :], preferred_element_type=jnp.float32))
    t2 = lin2 * dis[:, None]
    t2a_ref[...] = t2[:, :HALF]
    t2b_ref[...] = t2[:, HALF:]


def _mid_call(acc1a, acc1b, t1a, t1b, dis, b_c1, W_c2):
    n_blk = N_BLK
    return pl.pallas_call(
        _mid_block,
        grid=(n_blk,),
        in_specs=[
            pl.BlockSpec((R_BLK, HALF), lambda r: (r, 0)),
            pl.BlockSpec((R_BLK, HALF), lambda r: (r, 0)),
            pl.BlockSpec((R_BLK, HALF), lambda r: (r, 0)),
            pl.BlockSpec((R_BLK, HALF), lambda r: (r, 0)),
            pl.BlockSpec((R_BLK,), lambda r: (r,)),
            pl.BlockSpec((64,), lambda r: (0,)),
            pl.BlockSpec((64, 64), lambda r: (0, 0)),
        ],
        out_specs=[
            pl.BlockSpec((R_BLK, HALF), lambda r: (r, 0)),
            pl.BlockSpec((R_BLK, HALF), lambda r: (r, 0)),
        ],
        out_shape=[
            jax.ShapeDtypeStruct((N_NODES, HALF), jnp.float32),
            jax.ShapeDtypeStruct((N_NODES, HALF), jnp.float32),
        ],
    )(acc1a, acc1b, t1a, t1b, dis, b_c1, W_c2)


def _out_block(z_ref, wout_ref, bout_ref, o_ref):
    o_ref[...] = jnp.dot(z_ref[...], wout_ref[...],
                         preferred_element_type=jnp.float32) + bout_ref[...][None, :]


def _out_call(z, W_out, b_out):
    return pl.pallas_call(
        _out_block,
        grid=(4,),
        in_specs=[
            pl.BlockSpec((BATCH // 4, 64), lambda r: (r, 0)),
            pl.BlockSpec((64, 128), lambda r: (0, 0)),
            pl.BlockSpec((128,), lambda r: (0,)),
        ],
        out_specs=pl.BlockSpec((BATCH // 4, 128), lambda r: (r, 0)),
        out_shape=jax.ShapeDtypeStruct((BATCH, 128), jnp.float32),
    )(z, W_out, b_out)


@jax.jit
def kernel(x, edge_index, edge_attr, drug_indices,
           W_enc, b_enc, W_c1, b_c1, W_c2, b_c2, W_out, b_out):
    src = edge_index[0].astype(jnp.int32)
    dst = edge_index[1].astype(jnp.int32)
    ew = edge_attr.astype(jnp.float32)
    pad = E_PAD - src.shape[0]
    fill = (jnp.arange(pad, dtype=jnp.int32) * 67) % N_NODES
    src2d = jnp.concatenate([src, fill]).reshape(E_ROWS, 128)
    dst2d = jnp.concatenate([dst, fill]).reshape(E_ROWS, 128)
    ew1d = jnp.concatenate([ew, jnp.zeros((pad,), jnp.float32)])
    drug2d = drug_indices.astype(jnp.int32).reshape(32, 128)

    degp0, degp1 = _deg_call(dst2d, ew1d)
    t1a, t1b, dis = _enc_call(x, degp0, degp1, W_enc, b_enc, W_c1)
    acc1a, acc1b = _spmm_call(t1a, t1b, src2d, dst2d, ew1d)
    t2a, t2b = _mid_call(acc1a, acc1b, t1a, t1b, dis, b_c1, W_c2)
    acc2a, acc2b = _spmm_call(t2a, t2b, src2d, dst2d, ew1d)
    z = _gath_call(acc2a, acc2b, t2a, t2b, dis, b_c2, drug2d)
    return _out_call(z, W_out, b_out)


# speedup vs baseline: 2.9238x; 1.1191x over previous
import functools

import jax
import jax.numpy as jnp
import numpy as np
from jax import lax
from jax.experimental import pallas as pl
from jax.experimental.pallas import tpu as pltpu
from jax.experimental.pallas import tpu_sc as plsc

N_NODES = 50000
N_PAD = 50048
STRIPE = N_PAD // 16
HALF = 32
NC, NS = 2, 16

E_PAD = 811008
E_ROWS = E_PAD // 128
SP_TROWS = E_ROWS // 16
SP_CH = 3
SP_NBLK = SP_TROWS // SP_CH
DG_WROWS = E_ROWS // 32
DG_CH = 3
DG_NBLK = DG_WROWS // DG_CH

BATCH = 4096
R_BLK = 2048
N_BLK = 25

_MESH = plsc.VectorSubcoreMesh(
    core_axis_name="c", subcore_axis_name="s", num_cores=NC, num_subcores=NS)

def _shuffle_bf16(t):
    r = t.shape[0]
    return (t.reshape(r, 2, 16).transpose(0, 2, 1)
            .reshape(r, HALF).astype(jnp.bfloat16))


def _zero_rows(buf, n_rows, stripe_base, acc):
    def zb(e, _):
        buf[e, pl.ds(0, 16)] = jnp.zeros((16,), jnp.float32)
        buf[e, pl.ds(16, 16)] = jnp.zeros((16,), jnp.float32)
        return 0
    lax.fori_loop(0, n_rows, zb, 0)
    full, rem = STRIPE // n_rows, STRIPE % n_rows
    for k in range(full):
        pltpu.sync_copy(buf.at[pl.ds(0, n_rows)],
                        acc.at[pl.ds(stripe_base + k * n_rows, n_rows)])
    if rem:
        pltpu.sync_copy(buf.at[pl.ds(0, rem)],
                        acc.at[pl.ds(stripe_base + full * n_rows, rem)])


def _zero_1d(buf, n, stripe_base, acc):
    def zb(i, _):
        buf[pl.ds(i * 16, 16)] = jnp.zeros((16,), jnp.float32)
        return 0
    lax.fori_loop(0, n // 16, zb, 0)
    full, rem = STRIPE // n, STRIPE % n
    for k in range(full):
        pltpu.sync_copy(buf.at[pl.ds(0, n)],
                        acc.at[pl.ds(stripe_base + k * n, n)])
    if rem:
        pltpu.sync_copy(buf.at[pl.ds(0, rem)],
                        acc.at[pl.ds(stripe_base + full * n, rem)])


def _deg_body(dst2d, ew1d, out_p0, out_p1, acc, dbuf, ewbuf, zbuf, isem):
    c = lax.axis_index("c")
    s = lax.axis_index("s")
    _zero_1d(zbuf, 512, s * STRIPE, acc)
    plsc.subcore_barrier()
    w = c * NS + s
    NB = DG_NBLK

    def stage(b, p):
        row0 = w * DG_WROWS + b * DG_CH
        pltpu.async_copy(dst2d.at[pl.ds(row0, DG_CH)], dbuf.at[p], isem)
        pltpu.async_copy(ew1d.at[pl.ds(row0 * 128, DG_CH * 128)],
                         ewbuf.at[p], isem)

    def wait_stage(p):
        pltpu.make_async_copy(dst2d.at[pl.ds(0, DG_CH)], dbuf.at[p],
                              isem).wait()
        pltpu.make_async_copy(ew1d.at[pl.ds(0, DG_CH * 128)], ewbuf.at[p],
                              isem).wait()

    stage(0, 0)

    def blk(bb, _):
        for p in range(2):
            b = bb * 2 + p
            wait_stage(p)

            @pl.when(b + 1 < NB)
            def _():
                stage(b + 1, p ^ 1)
            for j in range(DG_CH):
                pltpu.sync_copy(ewbuf.at[p, pl.ds(j * 128, 128)],
                                acc.at[dbuf.at[p, j]], add=True)
        return 0

    lax.fori_loop(0, NB // 2, blk, 0)
    plsc.subcore_barrier()

    def drain(out):
        full, rem = STRIPE // 512, STRIPE % 512
        for k in range(full + (1 if rem else 0)):
            n = 512 if k < full else rem
            off = s * STRIPE + k * 512
            pltpu.sync_copy(acc.at[pl.ds(off, n)], zbuf.at[pl.ds(0, n)])
            pltpu.sync_copy(zbuf.at[pl.ds(0, n)], out.at[pl.ds(off, n)])

    @pl.when(c == 0)
    def _():
        drain(out_p0)

    @pl.when(c == 1)
    def _():
        drain(out_p1)


_deg_call = pl.kernel(
    _deg_body,
    out_type=[jax.ShapeDtypeStruct((N_PAD,), jnp.float32),
              jax.ShapeDtypeStruct((N_PAD,), jnp.float32)],
    mesh=_MESH,
    scratch_types=[
        pltpu.VMEM_SHARED((N_PAD,), jnp.float32),
        pltpu.VMEM((2, DG_CH, 128), jnp.int32),
        pltpu.VMEM((2, DG_CH * 128), jnp.float32),
        pltpu.VMEM((512,), jnp.float32),
        pltpu.SemaphoreType.DMA,
    ],
    compiler_params=pltpu.CompilerParams(use_tc_tiling_on_sc=False),
)


def _spmm_body(tab_a, tab_b, src2d, dst2d, ew1d, out_a, out_b,
               acc, sbuf, dbuf, ewbuf, rbuf, isem, gsem, ssem):
    c = lax.axis_index("c")
    s = lax.axis_index("s")
    _zero_rows(rbuf.at[0], SP_CH * 128, s * STRIPE, acc)
    plsc.subcore_barrier()
    NB = SP_NBLK

    def stage(b, p, q):
        r0 = s * SP_TROWS + b * SP_CH
        pltpu.async_copy(src2d.at[pl.ds(r0, SP_CH)], sbuf.at[p], isem)
        pltpu.async_copy(dst2d.at[pl.ds(r0, SP_CH)], dbuf.at[q], isem)
        pltpu.async_copy(ew1d.at[pl.ds(r0 * 128, SP_CH * 128)],
                         ewbuf.at[p], isem)

    def wait_stage(p, q):
        pltpu.make_async_copy(src2d.at[pl.ds(0, SP_CH)], sbuf.at[p],
                              isem).wait()
        pltpu.make_async_copy(dst2d.at[pl.ds(0, SP_CH)], dbuf.at[q],
                              isem).wait()
        pltpu.make_async_copy(ew1d.at[pl.ds(0, SP_CH * 128)], ewbuf.at[p],
                              isem).wait()

    def run(tab, out):
        def fire(p):
            for j in range(SP_CH):
                pltpu.async_copy(tab.at[sbuf.at[p, j]],
                                 rbuf.at[p, pl.ds(j * 128, 128)], gsem)

        def wait_fire(p):
            for j in range(SP_CH):
                pltpu.make_async_copy(tab.at[sbuf.at[p, j]],
                                      rbuf.at[p, pl.ds(j * 128, 128)],
                                      gsem).wait()

        def scale(p):
            def body(g, _):
                wv = ewbuf[p, pl.ds(g * 16, 16)]
                for k in range(16):
                    w = wv[k]
                    e = g * 16 + k
                    rbuf[p, e, pl.ds(0, 16)] = rbuf[p, e, pl.ds(0, 16)] * w
                    rbuf[p, e, pl.ds(16, 16)] = rbuf[p, e, pl.ds(16, 16)] * w
                return 0
            lax.fori_loop(0, SP_CH * 8, body, 0)

        def scat(p, q):
            for j in range(SP_CH):
                pltpu.async_copy(rbuf.at[p, pl.ds(j * 128, 128)],
                                 acc.at[dbuf.at[q, j]], ssem, add=True)

        def wait_scat(p, q):
            for j in range(SP_CH):
                pltpu.make_async_copy(rbuf.at[p, pl.ds(j * 128, 128)],
                                      acc.at[dbuf.at[q, j]], ssem).wait()

        stage(0, 0, 0)
        wait_stage(0, 0)
        fire(0)
        stage(1, 1, 1)

        def outer(bb, _):
            for q in range(4):
                b = bb * 4 + q
                p = q % 2

                @pl.when(b + 1 < NB)
                def _():
                    wait_stage(p ^ 1, (q + 1) % 4)

                @pl.when(b > 0)
                def _():
                    wait_scat(p ^ 1, (q - 1) % 4)

                @pl.when(b + 1 < NB)
                def _():
                    fire(p ^ 1)
                wait_fire(p)
                scale(p)
                scat(p, q)

                @pl.when(b + 2 < NB)
                def _():
                    stage(b + 2, p, (q + 2) % 4)
            return 0

        lax.fori_loop(0, NB // 4, outer, 0)
        wait_scat(1, 3)
        plsc.subcore_barrier()
        nbuf = SP_CH * 128
        full, rem = STRIPE // nbuf, STRIPE % nbuf
        for k in range(full + (1 if rem else 0)):
            n = nbuf if k < full else rem
            off = s * STRIPE + k * nbuf
            pltpu.sync_copy(acc.at[pl.ds(off, n)], rbuf.at[0, pl.ds(0, n)])
            pltpu.sync_copy(rbuf.at[0, pl.ds(0, n)], out.at[pl.ds(off, n)])

    @pl.when(c == 0)
    def _():
        run(tab_a, out_a)

    @pl.when(c == 1)
    def _():
        run(tab_b, out_b)


_spmm_call = pl.kernel(
    _spmm_body,
    out_type=[jax.ShapeDtypeStruct((N_PAD, HALF), jnp.float32),
              jax.ShapeDtypeStruct((N_PAD, HALF), jnp.float32)],
    mesh=_MESH,
    scratch_types=[
        pltpu.VMEM_SHARED((N_PAD, HALF), jnp.float32),
        pltpu.VMEM((2, SP_CH, 128), jnp.int32),
        pltpu.VMEM((4, SP_CH, 128), jnp.int32),
        pltpu.VMEM((2, SP_CH * 128), jnp.float32),
        pltpu.VMEM((2, SP_CH * 128, HALF), jnp.float32),
        pltpu.SemaphoreType.DMA,
        pltpu.SemaphoreType.DMA,
        pltpu.SemaphoreType.DMA,
    ],
    compiler_params=pltpu.CompilerParams(use_tc_tiling_on_sc=False),
)


def _gath_body(acc_a, acc_b, t2a, t2b, dis, bvec, drug2d, z_out,
               ibuf, ga, gb, ta, tb, db, bb, zbuf, sem):
    c = lax.axis_index("c")
    s = lax.axis_index("s")
    w = c * NS + s
    pltpu.sync_copy(drug2d.at[w], ibuf)
    pltpu.sync_copy(bvec, bb)
    descs = [
        pltpu.async_copy(acc_a.at[ibuf], ga, sem),
        pltpu.async_copy(acc_b.at[ibuf], gb, sem),
        pltpu.async_copy(t2a.at[ibuf], ta, sem),
        pltpu.async_copy(t2b.at[ibuf], tb, sem),
        pltpu.async_copy(dis.at[ibuf], db, sem),
    ]
    for d in descs:
        d.wait()
    b0 = bb[pl.ds(0, 16)]
    b1 = bb[pl.ds(16, 16)]
    b2 = bb[pl.ds(32, 16)]
    b3 = bb[pl.ds(48, 16)]

    def row(g, _):
        sv = db[pl.ds(g * 16, 16)]
        zero = jnp.zeros((16,), jnp.float32)
        for k in range(16):
            sc = sv[k]
            e = g * 16 + k
            ta0 = ta[e, pl.ds(0, 16)]
            ta1 = ta[e, pl.ds(16, 16)]
            tb0 = tb[e, pl.ds(0, 16)]
            tb1 = tb[e, pl.ds(16, 16)]
            zbuf[e, pl.ds(0, 16)] = jnp.maximum(
                (ga[e, pl.ds(0, 16)] + ta0) * sc + b0, zero)
            zbuf[e, pl.ds(16, 16)] = jnp.maximum(
                (ga[e, pl.ds(16, 16)] + ta1) * sc + b1, zero)
            zbuf[e, pl.ds(32, 16)] = jnp.maximum(
                (gb[e, pl.ds(0, 16)] + tb0) * sc + b2, zero)
            zbuf[e, pl.ds(48, 16)] = jnp.maximum(
                (gb[e, pl.ds(16, 16)] + tb1) * sc + b3, zero)
        return 0

    lax.fori_loop(0, 8, row, 0)
    pltpu.sync_copy(zbuf, z_out.at[pl.ds(w * 128, 128)])


_gath_call = pl.kernel(
    _gath_body,
    out_type=jax.ShapeDtypeStruct((BATCH, 2 * HALF), jnp.float32),
    mesh=_MESH,
    scratch_types=[
        pltpu.VMEM((128,), jnp.int32),
        pltpu.VMEM((128, HALF), jnp.float32),
        pltpu.VMEM((128, HALF), jnp.float32),
        pltpu.VMEM((128, HALF), jnp.float32),
        pltpu.VMEM((128, HALF), jnp.float32),
        pltpu.VMEM((128,), jnp.float32),
        pltpu.VMEM((64,), jnp.float32),
        pltpu.VMEM((128, 2 * HALF), jnp.float32),
        pltpu.SemaphoreType.DMA,
    ],
    compiler_params=pltpu.CompilerParams(use_tc_tiling_on_sc=False),
)


def _enc_block(x_ref, degp0_ref, degp1_ref, wenc_ref, benc_ref, wc1_ref,
               t1a_ref, t1b_ref, dis_ref):
    h0 = jnp.dot(x_ref[...], wenc_ref[...],
                 preferred_element_type=jnp.float32) + benc_ref[...][None, :]
    lin1 = jnp.dot(h0, wc1_ref[...], preferred_element_type=jnp.float32)
    deg = degp0_ref[...] + degp1_ref[...] + 1.0
    dis = lax.rsqrt(deg)
    t1 = lin1 * dis[:, None]
    t1a_ref[...] = t1[:, :HALF]
    t1b_ref[...] = t1[:, HALF:]
    dis_ref[...] = dis


def _enc_call(x, degp0, degp1, W_enc, b_enc, W_c1):
    n_blk = N_BLK
    return pl.pallas_call(
        _enc_block,
        grid=(n_blk,),
        in_specs=[
            pl.BlockSpec((R_BLK, 128), lambda r: (r, 0)),
            pl.BlockSpec((R_BLK,), lambda r: (r,)),
            pl.BlockSpec((R_BLK,), lambda r: (r,)),
            pl.BlockSpec((128, 64), lambda r: (0, 0)),
            pl.BlockSpec((64,), lambda r: (0,)),
            pl.BlockSpec((64, 64), lambda r: (0, 0)),
        ],
        out_specs=[
            pl.BlockSpec((R_BLK, HALF), lambda r: (r, 0)),
            pl.BlockSpec((R_BLK, HALF), lambda r: (r, 0)),
            pl.BlockSpec((R_BLK,), lambda r: (r,)),
        ],
        out_shape=[
            jax.ShapeDtypeStruct((N_NODES, HALF), jnp.float32),
            jax.ShapeDtypeStruct((N_NODES, HALF), jnp.float32),
            jax.ShapeDtypeStruct((N_NODES,), jnp.float32),
        ],
    )(x, degp0, degp1, W_enc, b_enc, W_c1)


def _mid_block(acc_a_ref, acc_b_ref, t1a_ref, t1b_ref, dis_ref, bc1_ref,
               wc2_ref, t2a_ref, t2b_ref):
    dis = dis_ref[...]
    b = bc1_ref[...]
    w = wc2_ref[...]
    ua = (acc_a_ref[...] + t1a_ref[...]) * dis[:, None] + b[None, :HALF]
    ub = (acc_b_ref[...] + t1b_ref[...]) * dis[:, None] + b[None, HALF:]
    h1a = jnp.maximum(ua, 0.0)
    h1b = jnp.maximum(ub, 0.0)
    lin2 = (jnp.dot(h1a, w[:HALF, :], preferred_element_type=jnp.float32)
            + jnp.dot(h1b, w[HALF:, :], preferred_element_type=jnp.float32))
    t2 = lin2 * dis[:, None]
    t2a_ref[...] = t2[:, :HALF]
    t2b_ref[...] = t2[:, HALF:]


def _mid_call(acc1a, acc1b, t1a, t1b, dis, b_c1, W_c2):
    n_blk = N_BLK
    return pl.pallas_call(
        _mid_block,
        grid=(n_blk,),
        in_specs=[
            pl.BlockSpec((R_BLK, HALF), lambda r: (r, 0)),
            pl.BlockSpec((R_BLK, HALF), lambda r: (r, 0)),
            pl.BlockSpec((R_BLK, HALF), lambda r: (r, 0)),
            pl.BlockSpec((R_BLK, HALF), lambda r: (r, 0)),
            pl.BlockSpec((R_BLK,), lambda r: (r,)),
            pl.BlockSpec((64,), lambda r: (0,)),
            pl.BlockSpec((64, 64), lambda r: (0, 0)),
        ],
        out_specs=[
            pl.BlockSpec((R_BLK, HALF), lambda r: (r, 0)),
            pl.BlockSpec((R_BLK, HALF), lambda r: (r, 0)),
        ],
        out_shape=[
            jax.ShapeDtypeStruct((N_NODES, HALF), jnp.float32),
            jax.ShapeDtypeStruct((N_NODES, HALF), jnp.float32),
        ],
    )(acc1a, acc1b, t1a, t1b, dis, b_c1, W_c2)


def _out_block(z_ref, wout_ref, bout_ref, o_ref):
    o_ref[...] = jnp.dot(z_ref[...], wout_ref[...],
                         preferred_element_type=jnp.float32) + bout_ref[...][None, :]


def _out_call(z, W_out, b_out):
    return pl.pallas_call(
        _out_block,
        grid=(4,),
        in_specs=[
            pl.BlockSpec((BATCH // 4, 64), lambda r: (r, 0)),
            pl.BlockSpec((64, 128), lambda r: (0, 0)),
            pl.BlockSpec((128,), lambda r: (0,)),
        ],
        out_specs=pl.BlockSpec((BATCH // 4, 128), lambda r: (r, 0)),
        out_shape=jax.ShapeDtypeStruct((BATCH, 128), jnp.float32),
    )(z, W_out, b_out)


@jax.jit
def kernel(x, edge_index, edge_attr, drug_indices,
           W_enc, b_enc, W_c1, b_c1, W_c2, b_c2, W_out, b_out):
    src = edge_index[0].astype(jnp.int32)
    dst = edge_index[1].astype(jnp.int32)
    ew = edge_attr.astype(jnp.float32)
    pad = E_PAD - src.shape[0]
    fill = (jnp.arange(pad, dtype=jnp.int32) * 67) % N_NODES
    src2d = jnp.concatenate([src, fill]).reshape(E_ROWS, 128)
    dst2d = jnp.concatenate([dst, fill]).reshape(E_ROWS, 128)
    ew1d = jnp.concatenate([ew, jnp.zeros((pad,), jnp.float32)])
    drug2d = drug_indices.astype(jnp.int32).reshape(32, 128)

    degp0, degp1 = _deg_call(dst2d, ew1d)
    t1a, t1b, dis = _enc_call(x, degp0, degp1, W_enc, b_enc, W_c1)
    acc1a, acc1b = _spmm_call(t1a, t1b, src2d, dst2d, ew1d)
    t2a, t2b = _mid_call(acc1a, acc1b, t1a, t1b, dis, b_c1, W_c2)
    acc2a, acc2b = _spmm_call(t2a, t2b, src2d, dst2d, ew1d)
    z = _gath_call(acc2a, acc2b, t2a, t2b, dis, b_c2, drug2d)
    return _out_call(z, W_out, b_out)
